# baseline (speedup 1.0000x reference)
import jax
import jax.numpy as jnp
from jax.experimental import pallas as pl
from jax.experimental.pallas import tpu as pltpu

_NUM_EXPERTS = 64
_TOP_K = 8
_MODEL_DIM = 2048
_BLOCK = 2048


def _router_kernel(x_ref, w_ref, out_ref):
    x = x_ref[...]
    w = w_ref[...]
    logits = jax.lax.dot_general(
        w, x, (((1,), (1,)), ((), ())), preferred_element_type=jnp.float32
    )
    m = jnp.max(logits, axis=0, keepdims=True)
    e = jnp.exp(logits - m)
    s = jnp.sum(e, axis=0, keepdims=True)
    probs = e / s
    out_ref[:, 0:_NUM_EXPERTS] = probs.T

    B = probs.shape[1]
    expert = jax.lax.broadcasted_iota(jnp.int32, (_NUM_EXPERTS, B), 0)
    pm = probs
    vals = []
    idxs = []
    for _ in range(_TOP_K):
        mj = jnp.max(pm, axis=0, keepdims=True)
        eq = pm == mj
        ij = jnp.min(jnp.where(eq, expert, _NUM_EXPERTS), axis=0,
                     keepdims=True)
        vals.append(mj)
        idxs.append(ij)
        pm = jnp.where(expert == ij, -jnp.inf, pm)
    v = jnp.concatenate(vals, axis=0)
    i = jnp.concatenate(idxs, axis=0)
    v = v / jnp.sum(v, axis=0, keepdims=True)
    out_ref[:, _NUM_EXPERTS:_NUM_EXPERTS + _TOP_K] = v.T
    out_ref[:, _NUM_EXPERTS + _TOP_K:_NUM_EXPERTS + 2 * _TOP_K] = (
        jax.lax.bitcast_convert_type(i.T, jnp.float32))
    out_ref[:, _NUM_EXPERTS + 2 * _TOP_K:] = jnp.zeros(
        (B, 128 - _NUM_EXPERTS - 2 * _TOP_K), jnp.float32)


def kernel(hidden_states, weight):
    x = hidden_states.reshape(-1, _MODEL_DIM)
    T = x.shape[0]
    packed = pl.pallas_call(
        _router_kernel,
        grid=(T // _BLOCK,),
        in_specs=[
            pl.BlockSpec((_BLOCK, _MODEL_DIM), lambda i: (i, 0)),
            pl.BlockSpec((_NUM_EXPERTS, _MODEL_DIM), lambda i: (0, 0)),
        ],
        out_specs=pl.BlockSpec((_BLOCK, 128), lambda i: (i, 0)),
        out_shape=jax.ShapeDtypeStruct((T, 128), jnp.float32),
        compiler_params=pltpu.CompilerParams(
            dimension_semantics=("arbitrary",),
        ),
    )(x, weight)
    probs = packed[:, :_NUM_EXPERTS]
    weights = packed[:, _NUM_EXPERTS:_NUM_EXPERTS + _TOP_K]
    idxs = jax.lax.bitcast_convert_type(
        packed[:, _NUM_EXPERTS + _TOP_K:_NUM_EXPERTS + 2 * _TOP_K], jnp.int32)
    return (probs, weights, idxs)


# speedup vs baseline: 1.7254x; 1.7254x over previous
import jax
import jax.numpy as jnp
from jax.experimental import pallas as pl
from jax.experimental.pallas import tpu as pltpu

_NUM_EXPERTS = 64
_TOP_K = 8
_MODEL_DIM = 2048
_BLOCK = 2048


def _router_kernel(x_ref, w_ref, probs_ref, weights_ref, idx_ref):
    x = x_ref[...]
    w = w_ref[...]
    logits = jax.lax.dot_general(
        w, x, (((1,), (1,)), ((), ())), preferred_element_type=jnp.float32
    )
    m = jnp.max(logits, axis=0, keepdims=True)
    e = jnp.exp(logits - m)
    s = jnp.sum(e, axis=0, keepdims=True)
    probs = e / s
    probs_ref[...] = probs

    B = probs.shape[1]
    expert = jax.lax.broadcasted_iota(jnp.int32, (_NUM_EXPERTS, B), 0)
    pm = probs
    vals = []
    idxs = []
    for _ in range(_TOP_K):
        mj = jnp.max(pm, axis=0, keepdims=True)
        eq = pm == mj
        ij = jnp.min(jnp.where(eq, expert, _NUM_EXPERTS), axis=0,
                     keepdims=True)
        vals.append(mj)
        idxs.append(ij)
        pm = jnp.where(expert == ij, -jnp.inf, pm)
    v = jnp.concatenate(vals, axis=0)
    i = jnp.concatenate(idxs, axis=0)
    weights_ref[...] = v / jnp.sum(v, axis=0, keepdims=True)
    idx_ref[...] = i


def kernel(hidden_states, weight):
    x = hidden_states.reshape(-1, _MODEL_DIM)
    T = x.shape[0]
    probs_t, weights_t, idx_t = pl.pallas_call(
        _router_kernel,
        grid=(T // _BLOCK,),
        in_specs=[
            pl.BlockSpec((_BLOCK, _MODEL_DIM), lambda i: (i, 0)),
            pl.BlockSpec((_NUM_EXPERTS, _MODEL_DIM), lambda i: (0, 0)),
        ],
        out_specs=[
            pl.BlockSpec((_NUM_EXPERTS, _BLOCK), lambda i: (0, i)),
            pl.BlockSpec((_TOP_K, _BLOCK), lambda i: (0, i)),
            pl.BlockSpec((_TOP_K, _BLOCK), lambda i: (0, i)),
        ],
        out_shape=[
            jax.ShapeDtypeStruct((_NUM_EXPERTS, T), jnp.float32),
            jax.ShapeDtypeStruct((_TOP_K, T), jnp.float32),
            jax.ShapeDtypeStruct((_TOP_K, T), jnp.int32),
        ],
        compiler_params=pltpu.CompilerParams(
            dimension_semantics=("arbitrary",),
        ),
    )(x, weight)
    return (probs_t.T, weights_t.T, idx_t.T)
